# Initial kernel scaffold; baseline (speedup 1.0000x reference)
import jax
import jax.numpy as jnp
from jax.experimental import pallas as pl


def kernel(edge_index, batch, rand_feat, W1, b1, W2, b2):
    raise NotImplementedError("write your pallas kernel here")


# speedup vs baseline: 111.6973x; 111.6973x over previous
import jax
import jax.numpy as jnp
from jax import lax
from jax.experimental import pallas as pl
from jax.experimental.pallas import tpu as pltpu
from jax.experimental.pallas import tpu_sc as plsc

N = 100000
E = 3200000
G = 1000
H = 16
NC, NS = 2, 16
NW = NC * NS
NPAD = 102400
GP = 1024
CH = 128
TOTCH = E // CH
CPW = TOTCH // NW
KG = 26
NGRP = 30
LEFTH = CPW - KG * NGRP
KG1 = 26
NGRP1 = 30
LEFT1 = CPW - KG1 * NGRP1
KGC = 12
NGRPC = 65
LEFTC = CPW - KGC * NGRPC
XCH = TOTCH - NW * CPW
F1 = 4
NSLICE = NPAD // NS
ZROWS = 400
NPW = NPAD // NW
CHF = 640
NTF = NPW // CHF
NSC = CHF // CH
GSL = GP // NS

_mesh = plsc.VectorSubcoreMesh(
    core_axis_name="c", subcore_axis_name="s", num_cores=NC, num_subcores=NS)

_f32 = jnp.float32


def _hist_body(ei, hs_out, hd_out, eidx_v, ones_v, zb1, hs_sh, hd_sh, sem):
    c = lax.axis_index("c")
    s = lax.axis_index("s")
    wid = c * NS + s
    z16 = jnp.zeros((16,), _f32)
    o16 = jnp.ones((16,), _f32)

    @pl.loop(0, NSLICE // 16)
    def _(i):
        zb1[pl.ds(i * 16, 16)] = z16

    @pl.loop(0, CH // 16)
    def _(i):
        ones_v[pl.ds(i * 16, 16)] = o16

    pltpu.sync_copy(zb1, hs_sh.at[pl.ds(s * NSLICE, NSLICE)])
    pltpu.sync_copy(zb1, hd_sh.at[pl.ds(s * NSLICE, NSLICE)])
    plsc.subcore_barrier()

    @pl.loop(0, NGRP)
    def _(g):
        c0 = wid + NW * (g * KG)
        ds_ = [pltpu.async_copy(ei.at[:, pl.ds((c0 + NW * b) * CH, CH)],
                                eidx_v.at[b], sem) for b in range(KG)]
        for d in ds_:
            d.wait()
        ds_ = []
        for b in range(KG):
            ds_.append(pltpu.async_copy(
                ones_v, hs_sh.at[eidx_v.at[b, 0]], sem, add=True))
            ds_.append(pltpu.async_copy(
                ones_v, hd_sh.at[eidx_v.at[b, 1]], sem, add=True))
        for d in ds_:
            d.wait()

    @pl.when(wid < XCH)
    def _():
        cid = TOTCH - XCH + wid
        pltpu.sync_copy(ei.at[:, pl.ds(cid * CH, CH)], eidx_v.at[0])
        pltpu.sync_copy(ones_v, hs_sh.at[eidx_v.at[0, 0]], add=True)
        pltpu.sync_copy(ones_v, hd_sh.at[eidx_v.at[0, 1]], add=True)

    plsc.subcore_barrier()
    pltpu.sync_copy(hs_sh.at[pl.ds(s * NSLICE, NSLICE)],
                    hs_out.at[c, pl.ds(s * NSLICE, NSLICE)])
    pltpu.sync_copy(hd_sh.at[pl.ds(s * NSLICE, NSLICE)],
                    hd_out.at[c, pl.ds(s * NSLICE, NSLICE)])


_hist = pl.kernel(
    _hist_body,
    out_type=[jax.ShapeDtypeStruct((NC, NPAD), _f32),
              jax.ShapeDtypeStruct((NC, NPAD), _f32)],
    mesh=_mesh,
    compiler_params=pltpu.CompilerParams(use_tc_tiling_on_sc=False),
    scratch_types=[
        pltpu.VMEM((KG, 2, CH), jnp.int32),
        pltpu.VMEM((CH,), _f32),
        pltpu.VMEM((NSLICE,), _f32),
        pltpu.VMEM_SHARED((NPAD,), _f32),
        pltpu.VMEM_SHARED((NPAD,), _f32),
        pltpu.SemaphoreType.DMA,
    ],
)


def _make_conv(fw, kg, ngrp, left):
    def body(ei, gtab, zrow, acc_out, eidx_v, rows_v, acc_sh, sem):
        c = lax.axis_index("c")
        s = lax.axis_index("s")
        wid = c * NS + s
        pltpu.sync_copy(zrow, rows_v.at[0])

        @pl.loop(0, NSLICE // CH)
        def _(i):
            pltpu.sync_copy(rows_v.at[0],
                            acc_sh.at[pl.ds(s * NSLICE + i * CH, CH)])

        plsc.subcore_barrier()

        @pl.loop(0, ngrp)
        def _(g):
            c0 = wid + NW * (g * kg)
            ds_ = [pltpu.async_copy(ei.at[:, pl.ds((c0 + NW * b) * CH, CH)],
                                    eidx_v.at[b], sem) for b in range(kg)]
            for d in ds_:
                d.wait()
            ds_ = [pltpu.async_copy(gtab.at[eidx_v.at[b, 0]], rows_v.at[b],
                                    sem) for b in range(kg)]
            for d in ds_:
                d.wait()
            ds_ = [pltpu.async_copy(rows_v.at[b], acc_sh.at[eidx_v.at[b, 1]],
                                    sem, add=True) for b in range(kg)]
            for d in ds_:
                d.wait()

        @pl.loop(0, left)
        def _(j):
            cid = wid + NW * (kg * ngrp + j)
            pltpu.sync_copy(ei.at[:, pl.ds(cid * CH, CH)], eidx_v.at[0])
            pltpu.sync_copy(gtab.at[eidx_v.at[0, 0]], rows_v.at[0])
            pltpu.sync_copy(rows_v.at[0], acc_sh.at[eidx_v.at[0, 1]],
                            add=True)

        @pl.when(wid < XCH)
        def _():
            cid = TOTCH - XCH + wid
            pltpu.sync_copy(ei.at[:, pl.ds(cid * CH, CH)], eidx_v.at[0])
            pltpu.sync_copy(gtab.at[eidx_v.at[0, 0]], rows_v.at[0])
            pltpu.sync_copy(rows_v.at[0], acc_sh.at[eidx_v.at[0, 1]],
                            add=True)

        plsc.subcore_barrier()
        pltpu.sync_copy(acc_sh.at[pl.ds(s * NSLICE, NSLICE)],
                        acc_out.at[c, pl.ds(s * NSLICE, NSLICE)])

    return pl.kernel(
        body,
        out_type=jax.ShapeDtypeStruct((NC, NPAD, fw), _f32),
        mesh=_mesh,
        compiler_params=pltpu.CompilerParams(use_tc_tiling_on_sc=False),
        scratch_types=[
            pltpu.VMEM((kg, 2, CH), jnp.int32),
            pltpu.VMEM((kg, CH, fw), _f32),
            pltpu.VMEM_SHARED((NPAD, fw), _f32),
            pltpu.SemaphoreType.DMA,
        ],
    )


_conv = _make_conv(H, KGC, NGRPC, LEFTC)


def _pool_body(acc2, g2t, dis, b2, bat3, pooled_out, counts_out,
               a0v, a1v, g2v, disv, b2v, bidx_v, ones_v, zb2, zb1,
               pooled_sh, counts_sh, sem):
    c = lax.axis_index("c")
    s = lax.axis_index("s")
    wid = c * NS + s
    z16 = jnp.zeros((16,), _f32)
    o16 = jnp.ones((16,), _f32)

    @pl.loop(0, GSL)
    def _(i):
        zb2[i, :] = z16

    @pl.loop(0, CH // 16)
    def _(i):
        zb1[pl.ds(i * 16, 16)] = z16
        ones_v[pl.ds(i * 16, 16)] = o16

    pltpu.sync_copy(b2, b2v)
    pltpu.sync_copy(zb2, pooled_sh.at[pl.ds(s * GSL, GSL)])

    @pl.when(s < GP // CH)
    def _():
        pltpu.sync_copy(zb1, counts_sh.at[pl.ds(s * CH, CH)])

    plsc.subcore_barrier()
    b2r = b2v[...]
    pltpu.sync_copy(bat3.at[pl.ds(wid * 32, NPW // CH)],
                    bidx_v.at[pl.ds(0, NPW // CH)])

    @pl.loop(0, NTF)
    def _(t):
        row0 = wid * NPW + t * CHF
        ds_ = [
            pltpu.async_copy(acc2.at[0, pl.ds(row0, CHF)], a0v, sem),
            pltpu.async_copy(acc2.at[1, pl.ds(row0, CHF)], a1v, sem),
            pltpu.async_copy(g2t.at[pl.ds(row0, CHF)], g2v, sem),
            pltpu.async_copy(dis.at[pl.ds(row0, CHF)], disv, sem),
        ]
        for d in ds_:
            d.wait()

        @pl.loop(0, CHF // 16)
        def _(jg):
            j0 = jg * 16
            dv = disv[pl.ds(j0, 16)]
            for l in range(16):
                j = j0 + l
                row = dv[l] * (a0v[j, :] + a1v[j, :] + g2v[j, :]) + b2r
                g2v[j, :] = row

        ds_ = []
        for k in range(NSC):
            ds_.append(pltpu.async_copy(
                g2v.at[pl.ds(k * CH, CH)], pooled_sh.at[bidx_v.at[t * NSC + k]],
                sem, add=True))
            ds_.append(pltpu.async_copy(
                ones_v, counts_sh.at[bidx_v.at[t * NSC + k]], sem, add=True))
        for d in ds_:
            d.wait()

    plsc.subcore_barrier()
    pltpu.sync_copy(pooled_sh.at[pl.ds(s * GSL, GSL)],
                    pooled_out.at[c, pl.ds(s * GSL, GSL)])

    @pl.when(s < GP // CH)
    def _():
        pltpu.sync_copy(counts_sh.at[pl.ds(s * CH, CH)],
                        counts_out.at[c, pl.ds(s * CH, CH)])


_pool = pl.kernel(
    _pool_body,
    out_type=[jax.ShapeDtypeStruct((NC, GP, H), _f32),
              jax.ShapeDtypeStruct((NC, GP), _f32)],
    mesh=_mesh,
    compiler_params=pltpu.CompilerParams(use_tc_tiling_on_sc=False),
    scratch_types=[
        pltpu.VMEM((CHF, H), _f32),
        pltpu.VMEM((CHF, H), _f32),
        pltpu.VMEM((CHF, H), _f32),
        pltpu.VMEM((CHF,), _f32),
        pltpu.VMEM((H,), _f32),
        pltpu.VMEM((32, CH), jnp.int32),
        pltpu.VMEM((CH,), _f32),
        pltpu.VMEM((GSL, H), _f32),
        pltpu.VMEM((CH,), _f32),
        pltpu.VMEM_SHARED((GP, H), _f32),
        pltpu.VMEM_SHARED((GP,), _f32),
        pltpu.SemaphoreType.DMA,
    ],
)


_R8 = NPAD // 8
_BR = 1600


def _tcb_body(hs0, hs1, hd0, hd1, rf8, k4, sel, g1u_o, dis8_o):
    hsum = hs0[...] + hs1[...]
    hdsum = hd0[...] + hd1[...]
    deg8 = hsum + hdsum
    dis8 = lax.rsqrt(hdsum + 1.0)
    dis8_o[...] = dis8
    k4v = k4[...]
    degrep = jnp.dot(deg8, k4v, preferred_element_type=_f32)
    rfrep = jnp.dot(rf8[...], k4v, preferred_element_type=_f32)
    disrep = jnp.dot(dis8, k4v, preferred_element_type=_f32)
    sv = sel[...]
    g1u_o[...] = disrep * (sv[0:1, :] + degrep * sv[1:2, :]
                           + rfrep * sv[2:3, :])


def _tc_b(hs0, hs1, hd0, hd1, rf8, k8, w1t):
    blk8 = pl.BlockSpec((_BR, 8), lambda i: (i, 0))
    blk128 = pl.BlockSpec((_BR, 128), lambda i: (i, 0))
    return pl.pallas_call(
        _tcb_body,
        grid=(8,),
        in_specs=[blk8, blk8, blk8, blk8, blk8,
                  pl.BlockSpec((8, 128), lambda i: (0, 0)),
                  pl.BlockSpec((3, 128), lambda i: (0, 0))],
        out_specs=[blk128, blk8],
        out_shape=[jax.ShapeDtypeStruct((_R8, 128), _f32),
                   jax.ShapeDtypeStruct((_R8, 8), _f32)],
    )(hs0, hs1, hd0, hd1, rf8, k8, w1t)


def _tcd_body(accr, uf8, dis8, w2k, k8, b1t, g2f_o):
    a = accr[...]
    g1f = a[0] + a[1] + uf8[...]
    disrep = jnp.dot(dis8[...], k8[...], preferred_element_type=_f32)
    out1 = disrep * g1f + b1t[...]
    r = jnp.maximum(out1, 0.0)
    h2 = jnp.dot(r, w2k[...], preferred_element_type=_f32)
    g2f_o[...] = disrep * h2


def _tc_d(accr, uf8, dis8, w2k, k8, b1t):
    blk8 = pl.BlockSpec((_BR, 8), lambda i: (i, 0))
    blk128 = pl.BlockSpec((_BR, 128), lambda i: (i, 0))
    return pl.pallas_call(
        _tcd_body,
        grid=(8,),
        in_specs=[pl.BlockSpec((NC, _BR, 128), lambda i: (0, i, 0)),
                  blk128, blk8,
                  pl.BlockSpec((128, 128), lambda i: (0, 0)),
                  pl.BlockSpec((8, 128), lambda i: (0, 0)),
                  pl.BlockSpec((1, 128), lambda i: (0, 0))],
        out_specs=blk128,
        out_shape=jax.ShapeDtypeStruct((_R8, 128), _f32),
    )(accr, uf8, dis8, w2k, k8, b1t)


def _tcg_body(p0, p1, c0, c1, out_o):
    pooled = p0[...] + p1[...]
    cnt = c0[...] + c1[...]
    mean = pooled / jnp.maximum(cnt, 1.0)
    m = jnp.max(mean, axis=1, keepdims=True)
    lse = jnp.log(jnp.sum(jnp.exp(mean - m), axis=1, keepdims=True)) + m
    out_o[...] = mean - lse


def _tc_g(p0, p1, c0, c1):
    full16 = pl.BlockSpec((GP, H), lambda: (0, 0))
    full1 = pl.BlockSpec((GP, 1), lambda: (0, 0))
    return pl.pallas_call(
        _tcg_body,
        in_specs=[full16, full16, full1, full1],
        out_specs=full16,
        out_shape=jax.ShapeDtypeStruct((GP, H), _f32),
    )(p0, p1, c0, c1)


def kernel(edge_index, batch, rand_feat, W1, b1, W2, b2):
    ei = edge_index
    hs, hd = _hist(ei)

    hs0 = hs[0].reshape(_R8, 8)
    hs1 = hs[1].reshape(_R8, 8)
    hd0 = hd[0].reshape(_R8, 8)
    hd1 = hd[1].reshape(_R8, 8)
    rf8 = jnp.pad(rand_feat[:, 0], (0, NPAD - N)).reshape(_R8, 8)
    k8 = jnp.kron(jnp.eye(8, dtype=_f32), jnp.ones((1, H), _f32))
    w1t = jnp.tile(W1, (1, 8))

    g1f, dis8 = _tc_b(hs0, hs1, hd0, hd1, rf8, k8, w1t)
    g1 = g1f.reshape(NPAD, H)

    acc1 = _conv(ei, g1, jnp.zeros((CH, H), _f32))

    w2k = jnp.kron(jnp.eye(8, dtype=_f32), W2)
    b1t = jnp.tile(b1, 8).reshape(1, 128)
    g2f = _tc_d(acc1.reshape(NC, _R8, 128), g1f, dis8, w2k, k8, b1t)
    g2 = g2f.reshape(NPAD, H)

    acc2 = _conv(ei, g2, jnp.zeros((CH, H), _f32))

    dis = dis8.reshape(NPAD)
    batch_pad = jnp.pad(batch, (0, NPAD - N), constant_values=GP - 1)
    bat3 = jnp.pad(batch_pad.reshape(NW, NPW // CH, CH),
                   ((0, 0), (0, 32 - NPW // CH), (0, 0))).reshape(NW * 32, CH)

    pooled, counts = _pool(acc2, g2, dis, b2, bat3)

    out = _tc_g(pooled[0], pooled[1],
                counts[0].reshape(GP, 1), counts[1].reshape(GP, 1))
    return out[:G]


# speedup vs baseline: 113.7587x; 1.0185x over previous
import jax
import jax.numpy as jnp
from jax import lax
from jax.experimental import pallas as pl
from jax.experimental.pallas import tpu as pltpu
from jax.experimental.pallas import tpu_sc as plsc

N = 100000
E = 3200000
G = 1000
H = 16
NC, NS = 2, 16
NW = NC * NS
NPAD = 102400
GP = 1024
CH = 256
TOTCH = E // CH
CPW = TOTCH // NW
KG = 26
NGRP = 15
LEFTH = CPW - KG * NGRP
KGC = 6
NGRPC = 65
LEFTC = CPW - KGC * NGRPC
XCH = TOTCH - NW * CPW
NSLICE = NPAD // NS
NPW = NPAD // NW
CHP = 128
CHF = 640
NTF = NPW // CHF
NSC = CHF // CHP
GSL = GP // NS

_mesh = plsc.VectorSubcoreMesh(
    core_axis_name="c", subcore_axis_name="s", num_cores=NC, num_subcores=NS)

_f32 = jnp.float32


def _hist_body(ei, hs_out, hd_out, eidx_v, ones_v, zb1, hs_sh, hd_sh, sem):
    c = lax.axis_index("c")
    s = lax.axis_index("s")
    wid = c * NS + s
    z16 = jnp.zeros((16,), _f32)
    o16 = jnp.ones((16,), _f32)

    @pl.loop(0, NSLICE // 16)
    def _(i):
        zb1[pl.ds(i * 16, 16)] = z16

    @pl.loop(0, CH // 16)
    def _(i):
        ones_v[pl.ds(i * 16, 16)] = o16

    pltpu.sync_copy(zb1, hs_sh.at[pl.ds(s * NSLICE, NSLICE)])
    pltpu.sync_copy(zb1, hd_sh.at[pl.ds(s * NSLICE, NSLICE)])
    plsc.subcore_barrier()

    @pl.loop(0, NGRP)
    def _(g):
        c0 = wid + NW * (g * KG)
        ds_ = [pltpu.async_copy(ei.at[:, pl.ds((c0 + NW * b) * CH, CH)],
                                eidx_v.at[b], sem) for b in range(KG)]
        for d in ds_:
            d.wait()
        ds_ = []
        for b in range(KG):
            ds_.append(pltpu.async_copy(
                ones_v, hs_sh.at[eidx_v.at[b, 0]], sem, add=True))
            ds_.append(pltpu.async_copy(
                ones_v, hd_sh.at[eidx_v.at[b, 1]], sem, add=True))
        for d in ds_:
            d.wait()

    @pl.when(wid < XCH)
    def _():
        cid = TOTCH - XCH + wid
        pltpu.sync_copy(ei.at[:, pl.ds(cid * CH, CH)], eidx_v.at[0])
        pltpu.sync_copy(ones_v, hs_sh.at[eidx_v.at[0, 0]], add=True)
        pltpu.sync_copy(ones_v, hd_sh.at[eidx_v.at[0, 1]], add=True)

    plsc.subcore_barrier()
    pltpu.sync_copy(hs_sh.at[pl.ds(s * NSLICE, NSLICE)],
                    hs_out.at[c, pl.ds(s * NSLICE, NSLICE)])
    pltpu.sync_copy(hd_sh.at[pl.ds(s * NSLICE, NSLICE)],
                    hd_out.at[c, pl.ds(s * NSLICE, NSLICE)])


_hist = pl.kernel(
    _hist_body,
    out_type=[jax.ShapeDtypeStruct((NC, NPAD), _f32),
              jax.ShapeDtypeStruct((NC, NPAD), _f32)],
    mesh=_mesh,
    compiler_params=pltpu.CompilerParams(use_tc_tiling_on_sc=False),
    scratch_types=[
        pltpu.VMEM((KG, 2, CH), jnp.int32),
        pltpu.VMEM((CH,), _f32),
        pltpu.VMEM((NSLICE,), _f32),
        pltpu.VMEM_SHARED((NPAD,), _f32),
        pltpu.VMEM_SHARED((NPAD,), _f32),
        pltpu.SemaphoreType.DMA,
    ],
)


def _make_conv(fw, kg, ngrp, left):
    def body(ei, gtab, zrow, acc_out, eidx_v, rows_v, acc_sh, sem):
        c = lax.axis_index("c")
        s = lax.axis_index("s")
        wid = c * NS + s
        pltpu.sync_copy(zrow, rows_v.at[0])

        @pl.loop(0, NSLICE // CH)
        def _(i):
            pltpu.sync_copy(rows_v.at[0],
                            acc_sh.at[pl.ds(s * NSLICE + i * CH, CH)])

        plsc.subcore_barrier()

        @pl.loop(0, ngrp)
        def _(g):
            c0 = wid + NW * (g * kg)
            ds_ = [pltpu.async_copy(ei.at[:, pl.ds((c0 + NW * b) * CH, CH)],
                                    eidx_v.at[b], sem) for b in range(kg)]
            for d in ds_:
                d.wait()
            ds_ = [pltpu.async_copy(gtab.at[eidx_v.at[b, 0]], rows_v.at[b],
                                    sem) for b in range(kg)]
            for d in ds_:
                d.wait()
            ds_ = [pltpu.async_copy(rows_v.at[b], acc_sh.at[eidx_v.at[b, 1]],
                                    sem, add=True) for b in range(kg)]
            for d in ds_:
                d.wait()

        @pl.loop(0, left)
        def _(j):
            cid = wid + NW * (kg * ngrp + j)
            pltpu.sync_copy(ei.at[:, pl.ds(cid * CH, CH)], eidx_v.at[0])
            pltpu.sync_copy(gtab.at[eidx_v.at[0, 0]], rows_v.at[0])
            pltpu.sync_copy(rows_v.at[0], acc_sh.at[eidx_v.at[0, 1]],
                            add=True)

        @pl.when(wid < XCH)
        def _():
            cid = TOTCH - XCH + wid
            pltpu.sync_copy(ei.at[:, pl.ds(cid * CH, CH)], eidx_v.at[0])
            pltpu.sync_copy(gtab.at[eidx_v.at[0, 0]], rows_v.at[0])
            pltpu.sync_copy(rows_v.at[0], acc_sh.at[eidx_v.at[0, 1]],
                            add=True)

        plsc.subcore_barrier()
        pltpu.sync_copy(acc_sh.at[pl.ds(s * NSLICE, NSLICE)],
                        acc_out.at[c, pl.ds(s * NSLICE, NSLICE)])

    return pl.kernel(
        body,
        out_type=jax.ShapeDtypeStruct((NC, NPAD, fw), _f32),
        mesh=_mesh,
        compiler_params=pltpu.CompilerParams(use_tc_tiling_on_sc=False),
        scratch_types=[
            pltpu.VMEM((kg, 2, CH), jnp.int32),
            pltpu.VMEM((kg, CH, fw), _f32),
            pltpu.VMEM_SHARED((NPAD, fw), _f32),
            pltpu.SemaphoreType.DMA,
        ],
    )


_conv = _make_conv(H, KGC, NGRPC, LEFTC)


def _pool_body(acc2, g2t, dis, b2, bat3, pooled_out, counts_out,
               a0v, a1v, g2v, disv, b2v, bidx_v, ones_v, zb2, zb1,
               pooled_sh, counts_sh, sem):
    c = lax.axis_index("c")
    s = lax.axis_index("s")
    wid = c * NS + s
    z16 = jnp.zeros((16,), _f32)
    o16 = jnp.ones((16,), _f32)

    @pl.loop(0, GSL)
    def _(i):
        zb2[i, :] = z16

    @pl.loop(0, CHP // 16)
    def _(i):
        zb1[pl.ds(i * 16, 16)] = z16
        ones_v[pl.ds(i * 16, 16)] = o16

    pltpu.sync_copy(b2, b2v)
    pltpu.sync_copy(zb2, pooled_sh.at[pl.ds(s * GSL, GSL)])

    @pl.when(s < GP // CHP)
    def _():
        pltpu.sync_copy(zb1, counts_sh.at[pl.ds(s * CHP, CHP)])

    plsc.subcore_barrier()
    b2r = b2v[...]
    pltpu.sync_copy(bat3.at[pl.ds(wid * 32, NPW // CHP)],
                    bidx_v.at[pl.ds(0, NPW // CHP)])

    @pl.loop(0, NTF)
    def _(t):
        row0 = wid * NPW + t * CHF
        ds_ = [
            pltpu.async_copy(acc2.at[0, pl.ds(row0, CHF)], a0v, sem),
            pltpu.async_copy(acc2.at[1, pl.ds(row0, CHF)], a1v, sem),
            pltpu.async_copy(g2t.at[pl.ds(row0, CHF)], g2v, sem),
            pltpu.async_copy(dis.at[pl.ds(row0, CHF)], disv, sem),
        ]
        for d in ds_:
            d.wait()

        @pl.loop(0, CHF // 16)
        def _(jg):
            j0 = jg * 16
            dv = disv[pl.ds(j0, 16)]
            for l in range(16):
                j = j0 + l
                row = dv[l] * (a0v[j, :] + a1v[j, :] + g2v[j, :]) + b2r
                g2v[j, :] = row

        ds_ = []
        for k in range(NSC):
            ds_.append(pltpu.async_copy(
                g2v.at[pl.ds(k * CHP, CHP)], pooled_sh.at[bidx_v.at[t * NSC + k]],
                sem, add=True))
            ds_.append(pltpu.async_copy(
                ones_v, counts_sh.at[bidx_v.at[t * NSC + k]], sem, add=True))
        for d in ds_:
            d.wait()

    plsc.subcore_barrier()
    pltpu.sync_copy(pooled_sh.at[pl.ds(s * GSL, GSL)],
                    pooled_out.at[c, pl.ds(s * GSL, GSL)])

    @pl.when(s < GP // CHP)
    def _():
        pltpu.sync_copy(counts_sh.at[pl.ds(s * CHP, CHP)],
                        counts_out.at[c, pl.ds(s * CHP, CHP)])


_pool = pl.kernel(
    _pool_body,
    out_type=[jax.ShapeDtypeStruct((NC, GP, H), _f32),
              jax.ShapeDtypeStruct((NC, GP), _f32)],
    mesh=_mesh,
    compiler_params=pltpu.CompilerParams(use_tc_tiling_on_sc=False),
    scratch_types=[
        pltpu.VMEM((CHF, H), _f32),
        pltpu.VMEM((CHF, H), _f32),
        pltpu.VMEM((CHF, H), _f32),
        pltpu.VMEM((CHF,), _f32),
        pltpu.VMEM((H,), _f32),
        pltpu.VMEM((32, CHP), jnp.int32),
        pltpu.VMEM((CHP,), _f32),
        pltpu.VMEM((GSL, H), _f32),
        pltpu.VMEM((CHP,), _f32),
        pltpu.VMEM_SHARED((GP, H), _f32),
        pltpu.VMEM_SHARED((GP,), _f32),
        pltpu.SemaphoreType.DMA,
    ],
)


_R8 = NPAD // 8
_BR = 1600


def _tcb_body(hs0, hs1, hd0, hd1, rf8, k4, sel, g1u_o, dis8_o):
    hsum = hs0[...] + hs1[...]
    hdsum = hd0[...] + hd1[...]
    deg8 = hsum + hdsum
    dis8 = lax.rsqrt(hdsum + 1.0)
    dis8_o[...] = dis8
    k4v = k4[...]
    degrep = jnp.dot(deg8, k4v, preferred_element_type=_f32)
    rfrep = jnp.dot(rf8[...], k4v, preferred_element_type=_f32)
    disrep = jnp.dot(dis8, k4v, preferred_element_type=_f32)
    sv = sel[...]
    g1u_o[...] = disrep * (sv[0:1, :] + degrep * sv[1:2, :]
                           + rfrep * sv[2:3, :])


def _tc_b(hs0, hs1, hd0, hd1, rf8, k8, w1t):
    blk8 = pl.BlockSpec((_BR, 8), lambda i: (i, 0))
    blk128 = pl.BlockSpec((_BR, 128), lambda i: (i, 0))
    return pl.pallas_call(
        _tcb_body,
        grid=(8,),
        in_specs=[blk8, blk8, blk8, blk8, blk8,
                  pl.BlockSpec((8, 128), lambda i: (0, 0)),
                  pl.BlockSpec((3, 128), lambda i: (0, 0))],
        out_specs=[blk128, blk8],
        out_shape=[jax.ShapeDtypeStruct((_R8, 128), _f32),
                   jax.ShapeDtypeStruct((_R8, 8), _f32)],
    )(hs0, hs1, hd0, hd1, rf8, k8, w1t)


def _tcd_body(accr, uf8, dis8, w2k, k8, b1t, g2f_o):
    a = accr[...]
    g1f = a[0] + a[1] + uf8[...]
    disrep = jnp.dot(dis8[...], k8[...], preferred_element_type=_f32)
    out1 = disrep * g1f + b1t[...]
    r = jnp.maximum(out1, 0.0)
    h2 = jnp.dot(r, w2k[...], preferred_element_type=_f32)
    g2f_o[...] = disrep * h2


def _tc_d(accr, uf8, dis8, w2k, k8, b1t):
    blk8 = pl.BlockSpec((_BR, 8), lambda i: (i, 0))
    blk128 = pl.BlockSpec((_BR, 128), lambda i: (i, 0))
    return pl.pallas_call(
        _tcd_body,
        grid=(8,),
        in_specs=[pl.BlockSpec((NC, _BR, 128), lambda i: (0, i, 0)),
                  blk128, blk8,
                  pl.BlockSpec((128, 128), lambda i: (0, 0)),
                  pl.BlockSpec((8, 128), lambda i: (0, 0)),
                  pl.BlockSpec((1, 128), lambda i: (0, 0))],
        out_specs=blk128,
        out_shape=jax.ShapeDtypeStruct((_R8, 128), _f32),
    )(accr, uf8, dis8, w2k, k8, b1t)


def _tcg_body(p0, p1, c0, c1, out_o):
    pooled = p0[...] + p1[...]
    cnt = c0[...] + c1[...]
    mean = pooled / jnp.maximum(cnt, 1.0)
    m = jnp.max(mean, axis=1, keepdims=True)
    lse = jnp.log(jnp.sum(jnp.exp(mean - m), axis=1, keepdims=True)) + m
    out_o[...] = mean - lse


def _tc_g(p0, p1, c0, c1):
    full16 = pl.BlockSpec((GP, H), lambda: (0, 0))
    full1 = pl.BlockSpec((GP, 1), lambda: (0, 0))
    return pl.pallas_call(
        _tcg_body,
        in_specs=[full16, full16, full1, full1],
        out_specs=full16,
        out_shape=jax.ShapeDtypeStruct((GP, H), _f32),
    )(p0, p1, c0, c1)


def kernel(edge_index, batch, rand_feat, W1, b1, W2, b2):
    ei = edge_index
    hs, hd = _hist(ei)

    hs0 = hs[0].reshape(_R8, 8)
    hs1 = hs[1].reshape(_R8, 8)
    hd0 = hd[0].reshape(_R8, 8)
    hd1 = hd[1].reshape(_R8, 8)
    rf8 = jnp.pad(rand_feat[:, 0], (0, NPAD - N)).reshape(_R8, 8)
    k8 = jnp.kron(jnp.eye(8, dtype=_f32), jnp.ones((1, H), _f32))
    w1t = jnp.tile(W1, (1, 8))

    g1f, dis8 = _tc_b(hs0, hs1, hd0, hd1, rf8, k8, w1t)
    g1 = g1f.reshape(NPAD, H)

    acc1 = _conv(ei, g1, jnp.zeros((CH, H), _f32))

    w2k = jnp.kron(jnp.eye(8, dtype=_f32), W2)
    b1t = jnp.tile(b1, 8).reshape(1, 128)
    g2f = _tc_d(acc1.reshape(NC, _R8, 128), g1f, dis8, w2k, k8, b1t)
    g2 = g2f.reshape(NPAD, H)

    acc2 = _conv(ei, g2, jnp.zeros((CH, H), _f32))

    dis = dis8.reshape(NPAD)
    batch_pad = jnp.pad(batch, (0, NPAD - N), constant_values=GP - 1)
    bat3 = jnp.pad(batch_pad.reshape(NW, NPW // CHP, CHP),
                   ((0, 0), (0, 32 - NPW // CHP), (0, 0))).reshape(NW * 32, CHP)

    pooled, counts = _pool(acc2, g2, dis, b2, bat3)

    out = _tc_g(pooled[0], pooled[1],
                counts[0].reshape(GP, 1), counts[1].reshape(GP, 1))
    return out[:G]


# speedup vs baseline: 133.7293x; 1.1756x over previous
import jax
import jax.numpy as jnp
from jax import lax
from jax.experimental import pallas as pl
from jax.experimental.pallas import tpu as pltpu
from jax.experimental.pallas import tpu_sc as plsc

N = 100000
E = 3200000
G = 1000
H = 16
NC, NS = 2, 16
NW = NC * NS
NPAD = 102400
GP = 1024
CH = 256
TOTCH = E // CH
CPW = TOTCH // NW
KG = 26
NGRP = 15
LEFTH = CPW - KG * NGRP
KGC = 6
NGRPC = 65
LEFTC = CPW - KGC * NGRPC
XCH = TOTCH - NW * CPW
NSLICE = NPAD // NS
NPW = NPAD // NW
CHP = 128
CHF = 640
NTF = NPW // CHF
NSC = CHF // CHP
GSL = GP // NS

_mesh = plsc.VectorSubcoreMesh(
    core_axis_name="c", subcore_axis_name="s", num_cores=NC, num_subcores=NS)

_f32 = jnp.float32


def _hist_body(ei, hs_out, hd_out, eidx_v, ones_v, zb1, hs_sh, hd_sh, sem,
               sem_s):
    c = lax.axis_index("c")
    s = lax.axis_index("s")
    wid = c * NS + s
    z16 = jnp.zeros((16,), _f32)
    o16 = jnp.ones((16,), _f32)

    @pl.loop(0, NSLICE // 16)
    def _(i):
        zb1[pl.ds(i * 16, 16)] = z16

    @pl.loop(0, CH // 16)
    def _(i):
        ones_v[pl.ds(i * 16, 16)] = o16

    pltpu.sync_copy(zb1, hs_sh.at[pl.ds(s * NSLICE, NSLICE)])
    pltpu.sync_copy(zb1, hd_sh.at[pl.ds(s * NSLICE, NSLICE)])
    plsc.subcore_barrier()

    @pl.loop(0, NGRP)
    def _(g):
        c0 = wid + NW * (g * KG)
        ds_i = [pltpu.async_copy(ei.at[:, pl.ds((c0 + NW * b) * CH, CH)],
                                 eidx_v.at[b], sem) for b in range(KG)]
        ds_ = []
        for b in range(KG):
            ds_i[b].wait()
            ds_.append(pltpu.async_copy(
                ones_v, hs_sh.at[eidx_v.at[b, 0]], sem_s, add=True))
            ds_.append(pltpu.async_copy(
                ones_v, hd_sh.at[eidx_v.at[b, 1]], sem_s, add=True))
        for d in ds_:
            d.wait()

    @pl.when(wid < XCH)
    def _():
        cid = TOTCH - XCH + wid
        pltpu.sync_copy(ei.at[:, pl.ds(cid * CH, CH)], eidx_v.at[0])
        pltpu.sync_copy(ones_v, hs_sh.at[eidx_v.at[0, 0]], add=True)
        pltpu.sync_copy(ones_v, hd_sh.at[eidx_v.at[0, 1]], add=True)

    plsc.subcore_barrier()
    pltpu.sync_copy(hs_sh.at[pl.ds(s * NSLICE, NSLICE)],
                    hs_out.at[c, pl.ds(s * NSLICE, NSLICE)])
    pltpu.sync_copy(hd_sh.at[pl.ds(s * NSLICE, NSLICE)],
                    hd_out.at[c, pl.ds(s * NSLICE, NSLICE)])


_hist = pl.kernel(
    _hist_body,
    out_type=[jax.ShapeDtypeStruct((NC, NPAD), _f32),
              jax.ShapeDtypeStruct((NC, NPAD), _f32)],
    mesh=_mesh,
    compiler_params=pltpu.CompilerParams(use_tc_tiling_on_sc=False),
    scratch_types=[
        pltpu.VMEM((KG, 2, CH), jnp.int32),
        pltpu.VMEM((CH,), _f32),
        pltpu.VMEM((NSLICE,), _f32),
        pltpu.VMEM_SHARED((NPAD,), _f32),
        pltpu.VMEM_SHARED((NPAD,), _f32),
        pltpu.SemaphoreType.DMA,
        pltpu.SemaphoreType.DMA,
    ],
)


def _make_conv(fw, kg, ngrp, left):
    def body(ei, gtab, zrow, acc_out, eidx_v, rows_v, acc_sh, sem, sem_g,
             sem_s):
        c = lax.axis_index("c")
        s = lax.axis_index("s")
        wid = c * NS + s
        pltpu.sync_copy(zrow, rows_v.at[0])

        @pl.loop(0, NSLICE // CH)
        def _(i):
            pltpu.sync_copy(rows_v.at[0],
                            acc_sh.at[pl.ds(s * NSLICE + i * CH, CH)])

        plsc.subcore_barrier()

        @pl.loop(0, ngrp)
        def _(g):
            c0 = wid + NW * (g * kg)
            ds_i = [pltpu.async_copy(ei.at[:, pl.ds((c0 + NW * b) * CH, CH)],
                                     eidx_v.at[b], sem) for b in range(kg)]
            ds_g = []
            for b in range(kg):
                ds_i[b].wait()
                ds_g.append(pltpu.async_copy(gtab.at[eidx_v.at[b, 0]],
                                             rows_v.at[b], sem_g))
            ds_s = []
            for b in range(kg):
                ds_g[b].wait()
                ds_s.append(pltpu.async_copy(rows_v.at[b],
                                             acc_sh.at[eidx_v.at[b, 1]],
                                             sem_s, add=True))
            for d in ds_s:
                d.wait()

        @pl.loop(0, left)
        def _(j):
            cid = wid + NW * (kg * ngrp + j)
            pltpu.sync_copy(ei.at[:, pl.ds(cid * CH, CH)], eidx_v.at[0])
            pltpu.sync_copy(gtab.at[eidx_v.at[0, 0]], rows_v.at[0])
            pltpu.sync_copy(rows_v.at[0], acc_sh.at[eidx_v.at[0, 1]],
                            add=True)

        @pl.when(wid < XCH)
        def _():
            cid = TOTCH - XCH + wid
            pltpu.sync_copy(ei.at[:, pl.ds(cid * CH, CH)], eidx_v.at[0])
            pltpu.sync_copy(gtab.at[eidx_v.at[0, 0]], rows_v.at[0])
            pltpu.sync_copy(rows_v.at[0], acc_sh.at[eidx_v.at[0, 1]],
                            add=True)

        plsc.subcore_barrier()
        pltpu.sync_copy(acc_sh.at[pl.ds(s * NSLICE, NSLICE)],
                        acc_out.at[c, pl.ds(s * NSLICE, NSLICE)])

    return pl.kernel(
        body,
        out_type=jax.ShapeDtypeStruct((NC, NPAD, fw), _f32),
        mesh=_mesh,
        compiler_params=pltpu.CompilerParams(use_tc_tiling_on_sc=False),
        scratch_types=[
            pltpu.VMEM((kg, 2, CH), jnp.int32),
            pltpu.VMEM((kg, CH, fw), _f32),
            pltpu.VMEM_SHARED((NPAD, fw), _f32),
            pltpu.SemaphoreType.DMA,
            pltpu.SemaphoreType.DMA,
            pltpu.SemaphoreType.DMA,
        ],
    )


_conv = _make_conv(H, KGC, NGRPC, LEFTC)


def _pool_body(acc2, g2t, dis, b2, bat3, pooled_out, counts_out,
               a0v, a1v, g2v, disv, b2v, bidx_v, ones_v, zb2, zb1,
               pooled_sh, counts_sh, sem):
    c = lax.axis_index("c")
    s = lax.axis_index("s")
    wid = c * NS + s
    z16 = jnp.zeros((16,), _f32)
    o16 = jnp.ones((16,), _f32)

    @pl.loop(0, GSL)
    def _(i):
        zb2[i, :] = z16

    @pl.loop(0, CHP // 16)
    def _(i):
        zb1[pl.ds(i * 16, 16)] = z16
        ones_v[pl.ds(i * 16, 16)] = o16

    pltpu.sync_copy(b2, b2v)
    pltpu.sync_copy(zb2, pooled_sh.at[pl.ds(s * GSL, GSL)])

    @pl.when(s < GP // CHP)
    def _():
        pltpu.sync_copy(zb1, counts_sh.at[pl.ds(s * CHP, CHP)])

    plsc.subcore_barrier()
    b2r = b2v[...]
    pltpu.sync_copy(bat3.at[pl.ds(wid * 32, NPW // CHP)],
                    bidx_v.at[pl.ds(0, NPW // CHP)])

    @pl.loop(0, NTF)
    def _(t):
        row0 = wid * NPW + t * CHF
        ds_ = [
            pltpu.async_copy(acc2.at[0, pl.ds(row0, CHF)], a0v, sem),
            pltpu.async_copy(acc2.at[1, pl.ds(row0, CHF)], a1v, sem),
            pltpu.async_copy(g2t.at[pl.ds(row0, CHF)], g2v, sem),
            pltpu.async_copy(dis.at[pl.ds(row0, CHF)], disv, sem),
        ]
        for d in ds_:
            d.wait()

        @pl.loop(0, CHF // 16)
        def _(jg):
            j0 = jg * 16
            dv = disv[pl.ds(j0, 16)]
            for l in range(16):
                j = j0 + l
                row = dv[l] * (a0v[j, :] + a1v[j, :] + g2v[j, :]) + b2r
                g2v[j, :] = row

        ds_ = []
        for k in range(NSC):
            ds_.append(pltpu.async_copy(
                g2v.at[pl.ds(k * CHP, CHP)], pooled_sh.at[bidx_v.at[t * NSC + k]],
                sem, add=True))
            ds_.append(pltpu.async_copy(
                ones_v, counts_sh.at[bidx_v.at[t * NSC + k]], sem, add=True))
        for d in ds_:
            d.wait()

    plsc.subcore_barrier()
    pltpu.sync_copy(pooled_sh.at[pl.ds(s * GSL, GSL)],
                    pooled_out.at[c, pl.ds(s * GSL, GSL)])

    @pl.when(s < GP // CHP)
    def _():
        pltpu.sync_copy(counts_sh.at[pl.ds(s * CHP, CHP)],
                        counts_out.at[c, pl.ds(s * CHP, CHP)])


_pool = pl.kernel(
    _pool_body,
    out_type=[jax.ShapeDtypeStruct((NC, GP, H), _f32),
              jax.ShapeDtypeStruct((NC, GP), _f32)],
    mesh=_mesh,
    compiler_params=pltpu.CompilerParams(use_tc_tiling_on_sc=False),
    scratch_types=[
        pltpu.VMEM((CHF, H), _f32),
        pltpu.VMEM((CHF, H), _f32),
        pltpu.VMEM((CHF, H), _f32),
        pltpu.VMEM((CHF,), _f32),
        pltpu.VMEM((H,), _f32),
        pltpu.VMEM((32, CHP), jnp.int32),
        pltpu.VMEM((CHP,), _f32),
        pltpu.VMEM((GSL, H), _f32),
        pltpu.VMEM((CHP,), _f32),
        pltpu.VMEM_SHARED((GP, H), _f32),
        pltpu.VMEM_SHARED((GP,), _f32),
        pltpu.SemaphoreType.DMA,
    ],
)


_R8 = NPAD // 8
_BR = 1600


def _tcb_body(hs0, hs1, hd0, hd1, rf8, k4, sel, g1u_o, dis8_o):
    hsum = hs0[...] + hs1[...]
    hdsum = hd0[...] + hd1[...]
    deg8 = hsum + hdsum
    dis8 = lax.rsqrt(hdsum + 1.0)
    dis8_o[...] = dis8
    k4v = k4[...]
    degrep = jnp.dot(deg8, k4v, preferred_element_type=_f32)
    rfrep = jnp.dot(rf8[...], k4v, preferred_element_type=_f32)
    disrep = jnp.dot(dis8, k4v, preferred_element_type=_f32)
    sv = sel[...]
    g1u_o[...] = disrep * (sv[0:1, :] + degrep * sv[1:2, :]
                           + rfrep * sv[2:3, :])


def _tc_b(hs0, hs1, hd0, hd1, rf8, k8, w1t):
    blk8 = pl.BlockSpec((_BR, 8), lambda i: (i, 0))
    blk128 = pl.BlockSpec((_BR, 128), lambda i: (i, 0))
    return pl.pallas_call(
        _tcb_body,
        grid=(8,),
        in_specs=[blk8, blk8, blk8, blk8, blk8,
                  pl.BlockSpec((8, 128), lambda i: (0, 0)),
                  pl.BlockSpec((3, 128), lambda i: (0, 0))],
        out_specs=[blk128, blk8],
        out_shape=[jax.ShapeDtypeStruct((_R8, 128), _f32),
                   jax.ShapeDtypeStruct((_R8, 8), _f32)],
    )(hs0, hs1, hd0, hd1, rf8, k8, w1t)


def _tcd_body(accr, uf8, dis8, w2k, k8, b1t, g2f_o):
    a = accr[...]
    g1f = a[0] + a[1] + uf8[...]
    disrep = jnp.dot(dis8[...], k8[...], preferred_element_type=_f32)
    out1 = disrep * g1f + b1t[...]
    r = jnp.maximum(out1, 0.0)
    h2 = jnp.dot(r, w2k[...], preferred_element_type=_f32)
    g2f_o[...] = disrep * h2


def _tc_d(accr, uf8, dis8, w2k, k8, b1t):
    blk8 = pl.BlockSpec((_BR, 8), lambda i: (i, 0))
    blk128 = pl.BlockSpec((_BR, 128), lambda i: (i, 0))
    return pl.pallas_call(
        _tcd_body,
        grid=(8,),
        in_specs=[pl.BlockSpec((NC, _BR, 128), lambda i: (0, i, 0)),
                  blk128, blk8,
                  pl.BlockSpec((128, 128), lambda i: (0, 0)),
                  pl.BlockSpec((8, 128), lambda i: (0, 0)),
                  pl.BlockSpec((1, 128), lambda i: (0, 0))],
        out_specs=blk128,
        out_shape=jax.ShapeDtypeStruct((_R8, 128), _f32),
    )(accr, uf8, dis8, w2k, k8, b1t)


def _tcg_body(p0, p1, c0, c1, out_o):
    pooled = p0[...] + p1[...]
    cnt = c0[...] + c1[...]
    mean = pooled / jnp.maximum(cnt, 1.0)
    m = jnp.max(mean, axis=1, keepdims=True)
    lse = jnp.log(jnp.sum(jnp.exp(mean - m), axis=1, keepdims=True)) + m
    out_o[...] = mean - lse


def _tc_g(p0, p1, c0, c1):
    full16 = pl.BlockSpec((GP, H), lambda: (0, 0))
    full1 = pl.BlockSpec((GP, 1), lambda: (0, 0))
    return pl.pallas_call(
        _tcg_body,
        in_specs=[full16, full16, full1, full1],
        out_specs=full16,
        out_shape=jax.ShapeDtypeStruct((GP, H), _f32),
    )(p0, p1, c0, c1)


def kernel(edge_index, batch, rand_feat, W1, b1, W2, b2):
    ei = edge_index
    hs, hd = _hist(ei)

    hs0 = hs[0].reshape(_R8, 8)
    hs1 = hs[1].reshape(_R8, 8)
    hd0 = hd[0].reshape(_R8, 8)
    hd1 = hd[1].reshape(_R8, 8)
    rf8 = jnp.pad(rand_feat[:, 0], (0, NPAD - N)).reshape(_R8, 8)
    k8 = jnp.kron(jnp.eye(8, dtype=_f32), jnp.ones((1, H), _f32))
    w1t = jnp.tile(W1, (1, 8))

    g1f, dis8 = _tc_b(hs0, hs1, hd0, hd1, rf8, k8, w1t)
    g1 = g1f.reshape(NPAD, H)

    acc1 = _conv(ei, g1, jnp.zeros((CH, H), _f32))

    w2k = jnp.kron(jnp.eye(8, dtype=_f32), W2)
    b1t = jnp.tile(b1, 8).reshape(1, 128)
    g2f = _tc_d(acc1.reshape(NC, _R8, 128), g1f, dis8, w2k, k8, b1t)
    g2 = g2f.reshape(NPAD, H)

    acc2 = _conv(ei, g2, jnp.zeros((CH, H), _f32))

    dis = dis8.reshape(NPAD)
    batch_pad = jnp.pad(batch, (0, NPAD - N), constant_values=GP - 1)
    bat3 = jnp.pad(batch_pad.reshape(NW, NPW // CHP, CHP),
                   ((0, 0), (0, 32 - NPW // CHP), (0, 0))).reshape(NW * 32, CHP)

    pooled, counts = _pool(acc2, g2, dis, b2, bat3)

    out = _tc_g(pooled[0], pooled[1],
                counts[0].reshape(GP, 1), counts[1].reshape(GP, 1))
    return out[:G]
